# Initial kernel scaffold; baseline (speedup 1.0000x reference)
import jax
import jax.numpy as jnp
from jax.experimental import pallas as pl


def kernel(q, v, embeddings):
    raise NotImplementedError("write your pallas kernel here")


# speedup vs baseline: 7.9502x; 7.9502x over previous
import functools

import jax
import jax.numpy as jnp
from jax.experimental import pallas as pl

BI = 128
BJ = 128


def _rpe_block(emb_ref, out_ref, *, mp, K, D, bi, bj):
    i0 = pl.program_id(0) * bi
    j0 = pl.program_id(1) * bj
    dmin = j0 - i0 - (bi - 1)
    dmax = j0 - i0 + (bj - 1)

    @pl.when(dmax <= -mp)
    def _low():
        out_ref[...] = jnp.broadcast_to(emb_ref[0, :], (bi, bj, D))

    @pl.when(dmin >= mp)
    def _high():
        out_ref[...] = jnp.broadcast_to(emb_ref[K - 1, :], (bi, bj, D))

    @pl.when(jnp.logical_and(dmax > -mp, dmin < mp))
    def _band():
        m = jax.lax.broadcasted_iota(jnp.int32, (bi * bj, K), 0)
        k = jax.lax.broadcasted_iota(jnp.int32, (bi * bj, K), 1)
        r = m // bj
        c = m - r * bj
        pos = jnp.clip((j0 + c) - (i0 + r), -mp, mp) + mp
        oh = (k == pos).astype(jnp.float32)
        res = jax.lax.dot_general(
            oh, emb_ref[...], (((1,), (0,)), ((), ())),
            preferred_element_type=jnp.float32)
        out_ref[...] = res.reshape(bi, bj, D)


def kernel(q, v, embeddings):
    sq, sv = q.shape[1], v.shape[1]
    K, D = embeddings.shape
    mp = (K - 1) // 2
    grid = (sq // BI, sv // BJ)
    return pl.pallas_call(
        functools.partial(_rpe_block, mp=mp, K=K, D=D, bi=BI, bj=BJ),
        grid=grid,
        in_specs=[pl.BlockSpec((K, D), lambda i, j: (0, 0))],
        out_specs=pl.BlockSpec((BI, BJ, D), lambda i, j: (i, j, 0)),
        out_shape=jax.ShapeDtypeStruct((sq, sv, D), jnp.float32),
    )(embeddings)
